# baseline (speedup 1.0000x reference)
import functools

import jax
import jax.numpy as jnp
from jax import lax
from jax.experimental import pallas as pl
from jax.experimental.pallas import tpu as pltpu
from jax.experimental.pallas import tpu_sc as plsc

_NUM_USERS = 1000000
_DIM = 32
_BATCH = 16384

_NC = 2
_NS = 16
_L = 16
_NW = _NC * _NS
_BPW = _BATCH // _NW
_CHUNK = 128
_NCHUNK = _BPW // _CHUNK

_mesh = plsc.VectorSubcoreMesh(core_axis_name="c", subcore_axis_name="s")


@functools.partial(
    pl.kernel,
    mesh=_mesh,
    out_type=jax.ShapeDtypeStruct((_BATCH, _DIM), jnp.float32),
    scratch_types=[
        pltpu.VMEM((_NCHUNK, _CHUNK), jnp.int32),
        pltpu.VMEM((_BPW, _DIM), jnp.float32),
        pltpu.SemaphoreType.DMA,
    ],
    compiler_params=pltpu.CompilerParams(use_tc_tiling_on_sc=False),
)
def _embed_lookup(ids_hbm, table_hbm, out_hbm, idx_v, rows_v, sem):
    wid = lax.axis_index("s") * _NC + lax.axis_index("c")
    base = wid * _BPW

    pltpu.sync_copy(ids_hbm.at[wid], idx_v)

    for j in range(_NCHUNK):
        for i in range(_CHUNK // _L):
            sl = pl.ds(i * _L, _L)
            v = idx_v[j, sl]
            idx_v[j, sl] = jnp.minimum(jnp.maximum(v, 0), _NUM_USERS)

    copies = [
        pltpu.async_copy(
            table_hbm.at[idx_v.at[j]],
            rows_v.at[pl.ds(j * _CHUNK, _CHUNK)],
            sem,
        )
        for j in range(_NCHUNK)
    ]
    for cp in copies:
        cp.wait()

    pltpu.sync_copy(rows_v, out_hbm.at[pl.ds(base, _BPW)])


def kernel(user_ids, table):
    ids = user_ids.astype(jnp.int32).reshape(_NW, _NCHUNK, _CHUNK)
    return _embed_lookup(ids, table)


# speedup vs baseline: 3.5491x; 3.5491x over previous
import functools

import jax
import jax.numpy as jnp
from jax import lax
from jax.experimental import pallas as pl
from jax.experimental.pallas import tpu as pltpu
from jax.experimental.pallas import tpu_sc as plsc

_NUM_USERS = 1000000
_DIM = 32
_BATCH = 16384

_NC = 2
_NS = 16
_L = 16
_NW = _NC * _NS
_BPW = _BATCH // _NW
_EPW = _BPW * _DIM
_WAVE = 8
_NWAVE = _BPW // _WAVE

_mesh = plsc.VectorSubcoreMesh(core_axis_name="c", subcore_axis_name="s")


@functools.partial(
    pl.kernel,
    mesh=_mesh,
    out_type=jax.ShapeDtypeStruct((_BATCH * _DIM,), jnp.float32),
    scratch_types=[
        pltpu.VMEM((_BPW,), jnp.int32),
        pltpu.VMEM((2, _WAVE, _DIM, 128), jnp.float32),
        pltpu.VMEM((_EPW,), jnp.float32),
        pltpu.SemaphoreType.DMA,
        pltpu.SemaphoreType.DMA,
    ],
    compiler_params=pltpu.CompilerParams(needs_layout_passes=False),
)
def _embed_lookup(
    ids_hbm, table_t_hbm, out_hbm, ids_v, blk_v, cols_v, sem0, sem1
):
    wid = lax.axis_index("s") * _NC + lax.axis_index("c")
    base = wid * _BPW
    sems = (sem0, sem1)

    pltpu.sync_copy(ids_hbm.at[pl.ds(base, _BPW)], ids_v)

    def load16(m):
        v = ids_v[pl.ds(m * _L, _L)]
        return jnp.minimum(jnp.maximum(v, 0), _NUM_USERS)

    def fire(v16, lane_base, slot):
        for i in range(_WAVE):
            u = v16[lane_base + i]
            ub = pl.multiple_of((u >> 7) << 7, 128)
            pltpu.make_async_copy(
                table_t_hbm.at[:, pl.ds(ub, 128)],
                blk_v.at[slot, i],
                sems[slot],
            ).start()

    def drain_extract(v16, lane_base, slot, w):
        for i in range(_WAVE):
            pltpu.make_async_copy(
                table_t_hbm.at[:, pl.ds(0, 128)],
                blk_v.at[slot, i],
                sems[slot],
            ).wait()
        rows = lax.iota(jnp.int32, _L)
        for i in range(_WAVE):
            ur = jnp.broadcast_to(v16[lane_base + i] & 127, (_L,))
            lo = plsc.load_gather(blk_v.at[slot, i], [rows, ur])
            hi = plsc.load_gather(blk_v.at[slot, i], [rows + _L, ur])
            k = w * _WAVE + i
            cols_v[pl.ds(k * _DIM, _L)] = lo
            cols_v[pl.ds(k * _DIM + _L, _L)] = hi

    fire(load16(0), 0, 0)

    def body(m, _):
        v16 = load16(m)
        fire(v16, _WAVE, 1)
        drain_extract(v16, 0, 0, 2 * m)

        @pl.when(m + 1 < _NWAVE // 2)
        def _prefetch():
            fire(load16(m + 1), 0, 0)

        drain_extract(v16, _WAVE, 1, 2 * m + 1)
        return _

    lax.fori_loop(0, _NWAVE // 2, body, 0)

    pltpu.sync_copy(cols_v, out_hbm.at[pl.ds(wid * _EPW, _EPW)])


def kernel(user_ids, table):
    ids = user_ids.astype(jnp.int32)
    flat = _embed_lookup(ids, table.T)
    return flat.reshape(_BATCH, _DIM)


# speedup vs baseline: 3.7777x; 1.0644x over previous
import functools

import jax
import jax.numpy as jnp
from jax import lax
from jax.experimental import pallas as pl
from jax.experimental.pallas import tpu as pltpu
from jax.experimental.pallas import tpu_sc as plsc

_NUM_USERS = 1000000
_DIM = 32
_BATCH = 16384

_NC = 2
_NS = 16
_L = 16
_NW = _NC * _NS
_BPW = _BATCH // _NW
_EPW = _BPW * _DIM
_WAVE = 8
_NWAVE = _BPW // _WAVE

_mesh = plsc.VectorSubcoreMesh(core_axis_name="c", subcore_axis_name="s")


@functools.partial(
    pl.kernel,
    mesh=_mesh,
    out_type=jax.ShapeDtypeStruct((_BATCH * _DIM,), jnp.float32),
    scratch_types=[
        pltpu.VMEM((_BPW,), jnp.int32),
        pltpu.VMEM((3, _WAVE, _DIM, 128), jnp.float32),
        pltpu.VMEM((_EPW,), jnp.float32),
        pltpu.SemaphoreType.DMA,
        pltpu.SemaphoreType.DMA,
        pltpu.SemaphoreType.DMA,
    ],
    compiler_params=pltpu.CompilerParams(needs_layout_passes=False),
)
def _embed_lookup(
    ids_hbm, table_t_hbm, out_hbm, ids_v, blk_v, cols_v, sem0, sem1, sem2
):
    wid = lax.axis_index("s") * _NC + lax.axis_index("c")
    base = wid * _BPW
    sems = (sem0, sem1, sem2)

    pltpu.sync_copy(ids_hbm.at[pl.ds(base, _BPW)], ids_v)

    def load16(m):
        v = ids_v[pl.ds(m * _L, _L)]
        return jnp.minimum(jnp.maximum(v, 0), _NUM_USERS)

    def fire(v16, lane_base, slot):
        for i in range(_WAVE):
            u = v16[lane_base + i]
            ub = pl.multiple_of((u >> 7) << 7, 128)
            pltpu.make_async_copy(
                table_t_hbm.at[:, pl.ds(ub, 128)],
                blk_v.at[slot, i],
                sems[slot],
            ).start()

    def drain_extract(v16, lane_base, slot, w):
        for i in range(_WAVE):
            pltpu.make_async_copy(
                table_t_hbm.at[:, pl.ds(0, 128)],
                blk_v.at[slot, i],
                sems[slot],
            ).wait()
        rows = lax.iota(jnp.int32, _L)
        for i in range(_WAVE):
            ur = jnp.broadcast_to(v16[lane_base + i] & 127, (_L,))
            lo = plsc.load_gather(blk_v.at[slot, i], [rows, ur])
            hi = plsc.load_gather(blk_v.at[slot, i], [rows + _L, ur])
            k = w * _WAVE + i
            cols_v[pl.ds(k * _DIM, _L)] = lo
            cols_v[pl.ds(k * _DIM + _L, _L)] = hi

    def fire_w(w, c):
        fire(load16(w // 2), (c % 2) * _WAVE, c % 3)

    def drain_w(w, c):
        drain_extract(load16(w // 2), (c % 2) * _WAVE, c % 3, w)

    fire_w(0, 0)
    fire_w(1, 1)

    def body(g, _):
        w0 = 6 * g
        for c in range(6):
            fire_w(w0 + c + 2, c + 2)
            drain_w(w0 + c, c)
        return _

    _NGRP = (_NWAVE - 4) // 6
    lax.fori_loop(0, _NGRP, body, 0)

    for w in range(_NWAVE - 4, _NWAVE):
        if w + 2 < _NWAVE:
            fire_w(w + 2, w + 2)
        drain_w(w, w)

    pltpu.sync_copy(cols_v, out_hbm.at[pl.ds(wid * _EPW, _EPW)])


def kernel(user_ids, table):
    ids = user_ids.astype(jnp.int32)
    flat = _embed_lookup(ids, table.T)
    return flat.reshape(_BATCH, _DIM)


# speedup vs baseline: 4.1012x; 1.0856x over previous
import functools

import jax
import jax.numpy as jnp
from jax import lax
from jax.experimental import pallas as pl
from jax.experimental.pallas import tpu as pltpu
from jax.experimental.pallas import tpu_sc as plsc

_NUM_USERS = 1000000
_DIM = 32
_BATCH = 16384

_NC = 2
_NS = 16
_L = 16
_NW = _NC * _NS
_BPW = _BATCH // _NW
_EPW = _BPW * _DIM
_WAVE = 8
_NWAVE = _BPW // _WAVE

_mesh = plsc.VectorSubcoreMesh(core_axis_name="c", subcore_axis_name="s")


@functools.partial(
    pl.kernel,
    mesh=_mesh,
    out_type=jax.ShapeDtypeStruct((_DIM, _BATCH), jnp.float32),
    scratch_types=[
        pltpu.VMEM((_BPW,), jnp.int32),
        pltpu.VMEM((3, _WAVE, _DIM, 128), jnp.float32),
        pltpu.VMEM((_DIM, _BPW), jnp.float32),
        pltpu.SemaphoreType.DMA,
        pltpu.SemaphoreType.DMA,
        pltpu.SemaphoreType.DMA,
    ],
    compiler_params=pltpu.CompilerParams(needs_layout_passes=False),
)
def _embed_lookup(
    ids_hbm, table_t_hbm, out_hbm, ids_v, blk_v, cols_v, sem0, sem1, sem2
):
    wid = lax.axis_index("s") * _NC + lax.axis_index("c")
    base = wid * _BPW
    sems = (sem0, sem1, sem2)

    pltpu.sync_copy(ids_hbm.at[pl.ds(base, _BPW)], ids_v)

    def load16(m):
        v = ids_v[pl.ds(m * _L, _L)]
        return jnp.minimum(jnp.maximum(v, 0), _NUM_USERS)

    def fire(v16, lane_base, slot):
        for i in range(_WAVE):
            u = v16[lane_base + i]
            ub = pl.multiple_of((u >> 7) << 7, 128)
            pltpu.make_async_copy(
                table_t_hbm.at[:, pl.ds(ub, 128)],
                blk_v.at[slot, i],
                sems[slot],
            ).start()

    def drain_extract(v16, lane_base, slot, w):
        for i in range(_WAVE):
            pltpu.make_async_copy(
                table_t_hbm.at[:, pl.ds(0, 128)],
                blk_v.at[slot, i],
                sems[slot],
            ).wait()
        rows = lax.iota(jnp.int32, _L)
        for i in range(_WAVE):
            ur = jnp.broadcast_to(v16[lane_base + i] & 127, (_L,))
            lo = plsc.load_gather(blk_v.at[slot, i], [rows, ur])
            hi = plsc.load_gather(blk_v.at[slot, i], [rows + _L, ur])
            k = jnp.broadcast_to(w * _WAVE + i, (_L,))
            plsc.store_scatter(cols_v, [rows, k], lo)
            plsc.store_scatter(cols_v, [rows + _L, k], hi)

    def fire_w(w, c):
        fire(load16(w // 2), (c % 2) * _WAVE, c % 3)

    def drain_w(w, c):
        drain_extract(load16(w // 2), (c % 2) * _WAVE, c % 3, w)

    fire_w(0, 0)
    fire_w(1, 1)

    def body(g, _):
        w0 = 6 * g
        for c in range(6):
            fire_w(w0 + c + 2, c + 2)
            drain_w(w0 + c, c)
        return _

    _NGRP = (_NWAVE - 4) // 6
    lax.fori_loop(0, _NGRP, body, 0)

    for w in range(_NWAVE - 4, _NWAVE):
        if w + 2 < _NWAVE:
            fire_w(w + 2, w + 2)
        drain_w(w, w)

    pltpu.sync_copy(cols_v, out_hbm.at[:, pl.ds(base, _BPW)])


def kernel(user_ids, table):
    ids = user_ids.astype(jnp.int32)
    out_t = _embed_lookup(ids, table.T)
    return out_t.T
